# baseline (speedup 1.0000x reference)
import functools

import jax
import jax.numpy as jnp
from jax.experimental import pallas as pl
from jax.experimental.pallas import tpu as pltpu

_K = 171
_IGNORE = 255
_M = 4096


def _loss_kernel(f_ref, lab_ref, p_ref, t_ref, loss_ref, acc_ref):
    i = pl.program_id(0)
    n_steps = pl.num_programs(0)

    @pl.when(i == 0)
    def _init():
        acc_ref[0] = 0.0
        acc_ref[1] = 0.0

    f = f_ref[0]
    lab = lab_ref[0]
    p = p_ref[...]
    t = t_ref[...]

    nrm2 = jnp.sum(f * f, axis=0, keepdims=True)
    inv_norm = jax.lax.rsqrt(jnp.maximum(nrm2, 1e-24))

    s = jax.lax.dot_general(
        p.astype(jnp.bfloat16), f.astype(jnp.bfloat16),
        (((1,), (0,)), ((), ())),
        preferred_element_type=jnp.float32,
    )

    inv_t = 1.0 / jnp.clip(t, 0.01, 1.0)
    logits = s * inv_norm * inv_t.T

    mx = jnp.max(logits, axis=0, keepdims=True)
    lse = jnp.log(jnp.sum(jnp.exp(logits - mx), axis=0, keepdims=True)) + mx

    safe_lab = jnp.clip(lab, 0, _K - 1)
    kiota = jax.lax.broadcasted_iota(jnp.int32, logits.shape, 0)
    picked = jnp.sum(jnp.where(kiota == safe_lab, logits, 0.0),
                     axis=0, keepdims=True)

    valid = (lab != _IGNORE).astype(jnp.float32)
    acc_ref[0] += jnp.sum((lse - picked) * valid)
    acc_ref[1] += jnp.sum(valid)

    @pl.when(i == n_steps - 1)
    def _final():
        loss_ref[...] = jnp.broadcast_to(
            acc_ref[0] / jnp.maximum(acc_ref[1], 1.0), (1, 1))


@functools.partial(jax.jit, static_argnames=())
def kernel(features, labels, prototypes, class_temperature):
    b, c, h, w = features.shape
    k = prototypes.shape[0]
    hw = h * w
    nm = hw // _M
    grid = b * nm

    feats = features.reshape(b, c, hw)
    labs = labels.reshape(grid, 1, _M)
    temps = class_temperature.reshape(1, k)

    loss = pl.pallas_call(
        _loss_kernel,
        grid=(grid,),
        in_specs=[
            pl.BlockSpec((1, c, _M), lambda i: (i // nm, 0, i % nm)),
            pl.BlockSpec((1, 1, _M), lambda i: (i, 0, 0)),
            pl.BlockSpec((k, c), lambda i: (0, 0)),
            pl.BlockSpec((1, k), lambda i: (0, 0)),
        ],
        out_specs=pl.BlockSpec((1, 1), lambda i: (0, 0)),
        out_shape=jax.ShapeDtypeStruct((1, 1), jnp.float32),
        scratch_shapes=[pltpu.SMEM((2,), jnp.float32)],
        compiler_params=pltpu.CompilerParams(
            dimension_semantics=("arbitrary",),
        ),
    )(feats, labs, prototypes, temps)

    return loss[0, 0]


# speedup vs baseline: 1.0111x; 1.0111x over previous
import functools

import jax
import jax.numpy as jnp
from jax.experimental import pallas as pl
from jax.experimental.pallas import tpu as pltpu

_K = 171
_IGNORE = 255
_M = 8192


def _loss_kernel(f_ref, lab_ref, p_ref, t_ref, loss_ref, acc_ref):
    i = pl.program_id(0)
    n_steps = pl.num_programs(0)

    @pl.when(i == 0)
    def _init():
        acc_ref[0] = 0.0
        acc_ref[1] = 0.0

    f = f_ref[0]
    lab = lab_ref[0]
    p = p_ref[...]
    t = t_ref[...]

    nrm2 = jnp.sum(f * f, axis=0, keepdims=True)
    inv_norm = jax.lax.rsqrt(jnp.maximum(nrm2, 1e-24))

    s = jax.lax.dot_general(
        p.astype(jnp.bfloat16), f.astype(jnp.bfloat16),
        (((1,), (0,)), ((), ())),
        preferred_element_type=jnp.float32,
    )

    inv_t = 1.0 / jnp.clip(t, 0.01, 1.0)
    logits = s * inv_norm * inv_t.T

    mx = jnp.max(logits, axis=0, keepdims=True)
    lse = jnp.log(jnp.sum(jnp.exp(logits - mx), axis=0, keepdims=True)) + mx

    safe_lab = jnp.clip(lab, 0, _K - 1)
    kiota = jax.lax.broadcasted_iota(jnp.int32, logits.shape, 0)
    picked = jnp.sum(jnp.where(kiota == safe_lab, logits, 0.0),
                     axis=0, keepdims=True)

    valid = (lab != _IGNORE).astype(jnp.float32)
    acc_ref[0] += jnp.sum((lse - picked) * valid)
    acc_ref[1] += jnp.sum(valid)

    @pl.when(i == n_steps - 1)
    def _final():
        loss_ref[...] = jnp.broadcast_to(
            acc_ref[0] / jnp.maximum(acc_ref[1], 1.0), (1, 1))


@functools.partial(jax.jit, static_argnames=())
def kernel(features, labels, prototypes, class_temperature):
    b, c, h, w = features.shape
    k = prototypes.shape[0]
    hw = h * w
    nm = hw // _M
    grid = b * nm

    feats = features.reshape(b, c, hw)
    labs = labels.reshape(grid, 1, _M)
    temps = class_temperature.reshape(1, k)

    loss = pl.pallas_call(
        _loss_kernel,
        grid=(grid,),
        in_specs=[
            pl.BlockSpec((1, c, _M), lambda i: (i // nm, 0, i % nm)),
            pl.BlockSpec((1, 1, _M), lambda i: (i, 0, 0)),
            pl.BlockSpec((k, c), lambda i: (0, 0)),
            pl.BlockSpec((1, k), lambda i: (0, 0)),
        ],
        out_specs=pl.BlockSpec((1, 1), lambda i: (0, 0)),
        out_shape=jax.ShapeDtypeStruct((1, 1), jnp.float32),
        scratch_shapes=[pltpu.SMEM((2,), jnp.float32)],
        compiler_params=pltpu.CompilerParams(
            dimension_semantics=("arbitrary",),
        ),
    )(feats, labs, prototypes, temps)

    return loss[0, 0]
